# Initial kernel scaffold; baseline (speedup 1.0000x reference)
import jax
import jax.numpy as jnp
from jax.experimental import pallas as pl


def kernel(ui_indices, ui_vals, iu_indices, iu_vals, user_id_emb, item_id_emb, image_feats, text_feats, W_img, b_img, W_txt, b_txt):
    raise NotImplementedError("write your pallas kernel here")


# speedup vs baseline: 1.7019x; 1.7019x over previous
import functools

import jax
import jax.numpy as jnp
from jax import lax
from jax.experimental import pallas as pl
from jax.experimental.pallas import tpu as pltpu
from jax.experimental.pallas import tpu_sc as plsc

_N = 50000
_D = 64
_HALF = _N // 2
_ACC = 25088
_RPT = _ACC // 16
_K = 80


def _proj_body(x_ref, w_ref, b_ref, o_ref):
    o_ref[...] = (
        jnp.dot(x_ref[...], w_ref[...], preferred_element_type=jnp.float32)
        + b_ref[...]
    )


def _project(x, W, b):
    M, F = x.shape
    BM = 2000
    return pl.pallas_call(
        _proj_body,
        grid=(M // BM,),
        in_specs=[
            pl.BlockSpec((BM, F), lambda i: (i, 0)),
            pl.BlockSpec((F, _D), lambda i: (0, 0)),
            pl.BlockSpec((1, _D), lambda i: (0, 0)),
        ],
        out_specs=pl.BlockSpec((BM, _D), lambda i: (i, 0)),
        out_shape=jax.ShapeDtypeStruct((M, _D), jnp.float32),
    )(x, W, b.reshape(1, _D))


def _stats_body(u_ref, i_ref, s1_ref, s2_ref):
    g = pl.program_id(0)

    @pl.when(g == 0)
    def _():
        s1_ref[...] = jnp.zeros_like(s1_ref)
        s2_ref[...] = jnp.zeros_like(s2_ref)

    xu = u_ref[...]
    xi = i_ref[...]
    s1_ref[...] += (jnp.sum(xu, axis=0) + jnp.sum(xi, axis=0)).reshape(1, _D)
    s2_ref[...] += (jnp.sum(xu * xu) + jnp.sum(xi * xi)).reshape(1, 1)


def _pairnorm_stats(u, i):
    BM = 5000
    return pl.pallas_call(
        _stats_body,
        grid=(_N // BM,),
        in_specs=[
            pl.BlockSpec((BM, _D), lambda g: (g, 0)),
            pl.BlockSpec((BM, _D), lambda g: (g, 0)),
        ],
        out_specs=[
            pl.BlockSpec((1, _D), lambda g: (0, 0)),
            pl.BlockSpec((1, 1), lambda g: (0, 0)),
        ],
        out_shape=[
            jax.ShapeDtypeStruct((1, _D), jnp.float32),
            jax.ShapeDtypeStruct((1, 1), jnp.float32),
        ],
    )(u, i)


def _pn_apply_body(u_ref, i_ref, s1_ref, s2_ref, uo_ref, io_ref):
    n = jnp.float32(2 * _N)
    mu = s1_ref[...] / n
    var = s2_ref[0, 0] / n - jnp.sum(mu * mu)
    inv = 1.0 / jnp.sqrt(var + 1e-6)
    uo_ref[...] = (u_ref[...] - mu) * inv
    io_ref[...] = (i_ref[...] - mu) * inv


def _pairnorm_apply(u, i, s1, s2):
    BM = 5000
    return pl.pallas_call(
        _pn_apply_body,
        grid=(_N // BM,),
        in_specs=[
            pl.BlockSpec((BM, _D), lambda g: (g, 0)),
            pl.BlockSpec((BM, _D), lambda g: (g, 0)),
            pl.BlockSpec((1, _D), lambda g: (0, 0)),
            pl.BlockSpec((1, 1), lambda g: (0, 0)),
        ],
        out_specs=[
            pl.BlockSpec((BM, _D), lambda g: (g, 0)),
            pl.BlockSpec((BM, _D), lambda g: (g, 0)),
        ],
        out_shape=[
            jax.ShapeDtypeStruct((_N, _D), jnp.float32),
            jax.ShapeDtypeStruct((_N, _D), jnp.float32),
        ],
    )(u, i, s1, s2)


def _combine_body(e0_ref, e1_ref, e2_ref, f1_ref, f2_ref, o_ref):
    def nrm(x):
        n = jnp.sqrt(jnp.sum(x * x, axis=1, keepdims=True))
        return x / jnp.maximum(n, 1e-12)

    base = (e0_ref[...] + e1_ref[...] + e2_ref[...]) * jnp.float32(1.0 / 3.0)
    o_ref[...] = base + 0.55 * nrm(f1_ref[...]) + 0.55 * nrm(f2_ref[...])


def _combine(e0, e1, e2, f1, f2):
    BM = 5000
    spec = pl.BlockSpec((BM, _D), lambda g: (g, 0))
    return pl.pallas_call(
        _combine_body,
        grid=(_N // BM,),
        in_specs=[spec] * 5,
        out_specs=spec,
        out_shape=jax.ShapeDtypeStruct((_N, _D), jnp.float32),
    )(e0, e1, e2, f1, f2)


def _sc_spmm_padded(dst, src, vals, x, zeros):
    E = dst.shape[0]
    ept = E // 16
    nch = ept // _K
    mesh = plsc.VectorSubcoreMesh(core_axis_name="c", subcore_axis_name="s")

    @functools.partial(
        pl.kernel,
        out_type=jax.ShapeDtypeStruct((2, _ACC, _D), jnp.float32),
        mesh=mesh,
        compiler_params=pltpu.CompilerParams(use_tc_tiling_on_sc=False),
        scratch_types=[
            pltpu.VMEM((_K,), jnp.int32),
            pltpu.VMEM((_K,), jnp.int32),
            pltpu.VMEM((_K,), jnp.float32),
            pltpu.VMEM((_K, _D), jnp.float32),
            pltpu.VMEM_SHARED((_ACC, _D), jnp.float32),
            pltpu.SemaphoreType.DMA,
        ],
    )
    def k(dst_hbm, src_hbm, val_hbm, x_hbm, z_hbm, out_hbm,
          dst_v, src_v, val_v, rows_v, acc, sem):
        c = lax.axis_index("c")
        s = lax.axis_index("s")
        lo = c * _HALF

        pltpu.sync_copy(z_hbm, acc.at[pl.ds(s * _RPT, _RPT)])
        plsc.subcore_barrier()

        def chunk(j, carry):
            base = s * ept + j * _K
            pltpu.sync_copy(dst_hbm.at[pl.ds(base, _K)], dst_v)
            pltpu.sync_copy(src_hbm.at[pl.ds(base, _K)], src_v)
            pltpu.sync_copy(val_hbm.at[pl.ds(base, _K)], val_v)
            for r in range(_K // 16):
                sl = pl.ds(r * 16, 16)
                d = dst_v[sl]
                m = (d >= lo) & (d < lo + _HALF)
                dst_v[sl] = jnp.where(m, d - lo, 0)
                val_v[sl] = jnp.where(m, val_v[sl], 0.0)
            pltpu.async_copy(x_hbm.at[src_v], rows_v, sem).wait()
            for blk in range(_K // 16):
                v16 = val_v[pl.ds(blk * 16, 16)]
                for t in range(16):
                    sv = jnp.broadcast_to(v16[t], (16,))
                    e = blk * 16 + t
                    for q in range(_D // 16):
                        rows_v[e, pl.ds(q * 16, 16)] = (
                            rows_v[e, pl.ds(q * 16, 16)] * sv)
            pltpu.sync_copy(rows_v, acc.at[dst_v], add=True)
            return carry

        lax.fori_loop(0, nch, chunk, 0)
        plsc.subcore_barrier()
        pltpu.sync_copy(acc.at[pl.ds(s * _RPT, _RPT)],
                        out_hbm.at[c, pl.ds(s * _RPT, _RPT)])

    return k(dst, src, vals, x, zeros)


def _sc_spmm(dst, src, vals, x, zeros):
    p = _sc_spmm_padded(dst, src, vals, x, zeros)
    return jnp.concatenate([p[0, :_HALF], p[1, :_HALF]], axis=0)


def kernel(ui_indices, ui_vals, iu_indices, iu_vals, user_id_emb, item_id_emb,
           image_feats, text_feats, W_img, b_img, W_txt, b_txt):
    image_feat = _project(image_feats, W_img, b_img)
    text_feat = _project(text_feats, W_txt, b_txt)

    s1, s2 = _pairnorm_stats(user_id_emb, item_id_emb)
    u0, i0 = _pairnorm_apply(user_id_emb, item_id_emb, s1, s2)

    ui_dst, ui_src = ui_indices[0], ui_indices[1]
    iu_dst, iu_src = iu_indices[0], iu_indices[1]
    zeros = jnp.zeros((_RPT, _D), jnp.float32)

    img_u = _sc_spmm(ui_dst, ui_src, ui_vals, image_feat, zeros)
    txt_u = _sc_spmm(ui_dst, ui_src, ui_vals, text_feat, zeros)
    u1 = _sc_spmm(ui_dst, ui_src, ui_vals, i0, zeros)
    img_i = _sc_spmm(iu_dst, iu_src, iu_vals, img_u, zeros)
    txt_i = _sc_spmm(iu_dst, iu_src, iu_vals, txt_u, zeros)
    i1 = _sc_spmm(iu_dst, iu_src, iu_vals, u1, zeros)
    u2 = _sc_spmm(ui_dst, ui_src, ui_vals, i1, zeros)
    i2 = _sc_spmm(iu_dst, iu_src, iu_vals, u2, zeros)

    u_final = _combine(u0, u1, u2, img_u, txt_u)
    i_final = _combine(i0, i1, i2, img_i, txt_i)
    return (u_final, i_final)


# speedup vs baseline: 4.2451x; 2.4943x over previous
import functools

import jax
import jax.numpy as jnp
from jax import lax
from jax.experimental import pallas as pl
from jax.experimental.pallas import tpu as pltpu
from jax.experimental.pallas import tpu_sc as plsc

_N = 50000
_D = 64
_HALF = _N // 2
_ACC = 25088
_RPT = _ACC // 16
_K = 80
_SUP = 2000
_NCH = _SUP // _K


def _proj_body(x_ref, w_ref, b_ref, o_ref):
    o_ref[...] = (
        jnp.dot(x_ref[...], w_ref[...], preferred_element_type=jnp.float32)
        + b_ref[...]
    )


def _project(x, W, b):
    M, F = x.shape
    BM = 2000
    return pl.pallas_call(
        _proj_body,
        grid=(M // BM,),
        in_specs=[
            pl.BlockSpec((BM, F), lambda i: (i, 0)),
            pl.BlockSpec((F, _D), lambda i: (0, 0)),
            pl.BlockSpec((1, _D), lambda i: (0, 0)),
        ],
        out_specs=pl.BlockSpec((BM, _D), lambda i: (i, 0)),
        out_shape=jax.ShapeDtypeStruct((M, _D), jnp.float32),
    )(x, W, b.reshape(1, _D))


def _stats_body(u_ref, i_ref, s1_ref, s2_ref):
    g = pl.program_id(0)

    @pl.when(g == 0)
    def _():
        s1_ref[...] = jnp.zeros_like(s1_ref)
        s2_ref[...] = jnp.zeros_like(s2_ref)

    xu = u_ref[...]
    xi = i_ref[...]
    s1_ref[...] += (jnp.sum(xu, axis=0) + jnp.sum(xi, axis=0)).reshape(1, _D)
    s2_ref[...] += (jnp.sum(xu * xu) + jnp.sum(xi * xi)).reshape(1, 1)


def _pairnorm_stats(u, i):
    BM = 5000
    return pl.pallas_call(
        _stats_body,
        grid=(_N // BM,),
        in_specs=[
            pl.BlockSpec((BM, _D), lambda g: (g, 0)),
            pl.BlockSpec((BM, _D), lambda g: (g, 0)),
        ],
        out_specs=[
            pl.BlockSpec((1, _D), lambda g: (0, 0)),
            pl.BlockSpec((1, 1), lambda g: (0, 0)),
        ],
        out_shape=[
            jax.ShapeDtypeStruct((1, _D), jnp.float32),
            jax.ShapeDtypeStruct((1, 1), jnp.float32),
        ],
    )(u, i)


def _pn_apply_body(u_ref, i_ref, s1_ref, s2_ref, uo_ref, io_ref):
    n = jnp.float32(2 * _N)
    mu = s1_ref[...] / n
    var = s2_ref[0, 0] / n - jnp.sum(mu * mu)
    inv = 1.0 / jnp.sqrt(var + 1e-6)
    uo_ref[...] = (u_ref[...] - mu) * inv
    io_ref[...] = (i_ref[...] - mu) * inv


def _pairnorm_apply(u, i, s1, s2):
    BM = 5000
    return pl.pallas_call(
        _pn_apply_body,
        grid=(_N // BM,),
        in_specs=[
            pl.BlockSpec((BM, _D), lambda g: (g, 0)),
            pl.BlockSpec((BM, _D), lambda g: (g, 0)),
            pl.BlockSpec((1, _D), lambda g: (0, 0)),
            pl.BlockSpec((1, 1), lambda g: (0, 0)),
        ],
        out_specs=[
            pl.BlockSpec((BM, _D), lambda g: (g, 0)),
            pl.BlockSpec((BM, _D), lambda g: (g, 0)),
        ],
        out_shape=[
            jax.ShapeDtypeStruct((_N, _D), jnp.float32),
            jax.ShapeDtypeStruct((_N, _D), jnp.float32),
        ],
    )(u, i, s1, s2)


def _combine_body(e0_ref, e1_ref, e2_ref, f1_ref, f2_ref, o_ref):
    def nrm(x):
        n = jnp.sqrt(jnp.sum(x * x, axis=1, keepdims=True))
        return x / jnp.maximum(n, 1e-12)

    base = (e0_ref[...] + e1_ref[...] + e2_ref[...]) * jnp.float32(1.0 / 3.0)
    o_ref[...] = base + 0.55 * nrm(f1_ref[...]) + 0.55 * nrm(f2_ref[...])


def _combine(e0, e1, e2, f1, f2):
    BM = 5000
    spec = pl.BlockSpec((BM, _D), lambda g: (g, 0))
    return pl.pallas_call(
        _combine_body,
        grid=(_N // BM,),
        in_specs=[spec] * 5,
        out_specs=spec,
        out_shape=jax.ShapeDtypeStruct((_N, _D), jnp.float32),
    )(e0, e1, e2, f1, f2)


def _sc_spmm_padded(dst, src, vals, x, zeros):
    E = dst.shape[0]
    ept = E // 16
    nsup = ept // _SUP
    mesh = plsc.VectorSubcoreMesh(core_axis_name="c", subcore_axis_name="s")

    @functools.partial(
        pl.kernel,
        out_type=jax.ShapeDtypeStruct((2, _ACC, _D), jnp.float32),
        mesh=mesh,
        compiler_params=pltpu.CompilerParams(use_tc_tiling_on_sc=False),
        scratch_types=[
            pltpu.VMEM((_SUP,), jnp.int32),
            pltpu.VMEM((_SUP,), jnp.int32),
            pltpu.VMEM((_SUP,), jnp.float32),
            pltpu.VMEM((_K,), jnp.int32),
            pltpu.VMEM((_K, _D), jnp.float32),
            pltpu.VMEM((_K, _D), jnp.float32),
            pltpu.VMEM_SHARED((_ACC, _D), jnp.float32),
            pltpu.SemaphoreType.DMA,
            pltpu.SemaphoreType.DMA,
        ],
    )
    def k(dst_hbm, src_hbm, val_hbm, x_hbm, z_hbm, out_hbm,
          dst_s, src_s, val_s, idx_w, rows0, rows1, acc, sem0, sem1):
        c = lax.axis_index("c")
        s = lax.axis_index("s")
        lo = c * _HALF
        hi = lo + _HALF

        pltpu.sync_copy(z_hbm, acc.at[pl.ds(s * _RPT, _RPT)])
        plsc.subcore_barrier()

        def issue(cb, buf, sem):
            pltpu.async_copy(x_hbm.at[src_s.at[pl.ds(cb, _K)]], buf, sem)

        def drain(buf, sem):
            pltpu.make_async_copy(x_hbm.at[pl.ds(0, _K)], buf, sem).wait()

        def process(cb, buf):
            for t in range(_K // 16):
                idx_w[pl.ds(t * 16, 16)] = dst_s[pl.ds(cb + t * 16, 16)]

            def scale_blk(blk, cc):
                v16 = val_s[pl.ds(cb + blk * 16, 16)]
                row = blk * 16
                for t in range(16):
                    sv = jnp.broadcast_to(v16[t], (16,))
                    for q in range(_D // 16):
                        buf[row + t, pl.ds(q * 16, 16)] = (
                            buf[row + t, pl.ds(q * 16, 16)] * sv)
                return cc

            lax.fori_loop(0, _K // 16, scale_blk, 0)
            pltpu.sync_copy(buf, acc.at[idx_w], add=True)

        def super_body(sup, carry):
            ebase = s * ept + sup * _SUP
            pltpu.sync_copy(dst_hbm.at[pl.ds(ebase, _SUP)], dst_s)
            pltpu.sync_copy(src_hbm.at[pl.ds(ebase, _SUP)], src_s)
            pltpu.sync_copy(val_hbm.at[pl.ds(ebase, _SUP)], val_s)

            def filt(r, cc):
                sl = pl.ds(r * 16, 16)
                d = dst_s[sl]
                m = (d >= lo) & (d < hi)
                dst_s[sl] = jnp.where(m, d - lo, 0)
                val_s[sl] = jnp.where(m, val_s[sl], 0.0)
                return cc

            lax.fori_loop(0, _SUP // 16, filt, 0)

            issue(0, rows0, sem0)

            def pair(p, cc):
                c0 = 2 * p * _K
                c1 = c0 + _K
                i1 = 2 * p + 1

                @pl.when(i1 < _NCH)
                def _():
                    issue(c1, rows1, sem1)

                drain(rows0, sem0)
                process(c0, rows0)

                @pl.when(i1 + 1 < _NCH)
                def _():
                    issue(c1 + _K, rows0, sem0)

                @pl.when(i1 < _NCH)
                def _():
                    drain(rows1, sem1)
                    process(c1, rows1)

                return cc

            lax.fori_loop(0, (_NCH + 1) // 2, pair, 0)
            return carry

        lax.fori_loop(0, nsup, super_body, 0)
        plsc.subcore_barrier()
        pltpu.sync_copy(acc.at[pl.ds(s * _RPT, _RPT)],
                        out_hbm.at[c, pl.ds(s * _RPT, _RPT)])

    return k(dst, src, vals, x, zeros)


def _sc_spmm(dst, src, vals, x, zeros):
    p = _sc_spmm_padded(dst, src, vals, x, zeros)
    return jnp.concatenate([p[0, :_HALF], p[1, :_HALF]], axis=0)


def kernel(ui_indices, ui_vals, iu_indices, iu_vals, user_id_emb, item_id_emb,
           image_feats, text_feats, W_img, b_img, W_txt, b_txt):
    image_feat = _project(image_feats, W_img, b_img)
    text_feat = _project(text_feats, W_txt, b_txt)

    s1, s2 = _pairnorm_stats(user_id_emb, item_id_emb)
    u0, i0 = _pairnorm_apply(user_id_emb, item_id_emb, s1, s2)

    ui_dst, ui_src = ui_indices[0], ui_indices[1]
    iu_dst, iu_src = iu_indices[0], iu_indices[1]
    zeros = jnp.zeros((_RPT, _D), jnp.float32)

    img_u = _sc_spmm(ui_dst, ui_src, ui_vals, image_feat, zeros)
    txt_u = _sc_spmm(ui_dst, ui_src, ui_vals, text_feat, zeros)
    u1 = _sc_spmm(ui_dst, ui_src, ui_vals, i0, zeros)
    img_i = _sc_spmm(iu_dst, iu_src, iu_vals, img_u, zeros)
    txt_i = _sc_spmm(iu_dst, iu_src, iu_vals, txt_u, zeros)
    i1 = _sc_spmm(iu_dst, iu_src, iu_vals, u1, zeros)
    u2 = _sc_spmm(ui_dst, ui_src, ui_vals, i1, zeros)
    i2 = _sc_spmm(iu_dst, iu_src, iu_vals, u2, zeros)

    u_final = _combine(u0, u1, u2, img_u, txt_u)
    i_final = _combine(i0, i1, i2, img_i, txt_i)
    return (u_final, i_final)


# speedup vs baseline: 5.7156x; 1.3464x over previous
import functools

import jax
import jax.numpy as jnp
from jax import lax
from jax.experimental import pallas as pl
from jax.experimental.pallas import tpu as pltpu
from jax.experimental.pallas import tpu_sc as plsc

_N = 50000
_D = 64
_H = 32
_R = 50176
_RPT = _R // 16
_K = 80
_SUP = 2000
_NCH = _SUP // _K


def _proj_body(x_ref, w_ref, b_ref, o_ref):
    res = (
        jnp.dot(x_ref[...], w_ref[...], preferred_element_type=jnp.float32)
        + b_ref[...]
    )
    o_ref[0] = res[:, :_H]
    o_ref[1] = res[:, _H:]


def _project(x, W, b):
    BM = 1000
    return pl.pallas_call(
        _proj_body,
        grid=(_N // BM,),
        in_specs=[
            pl.BlockSpec((BM, 128), lambda g: (g, 0)),
            pl.BlockSpec((128, _D), lambda g: (0, 0)),
            pl.BlockSpec((1, _D), lambda g: (0, 0)),
        ],
        out_specs=pl.BlockSpec((2, BM, _H), lambda g: (0, g, 0)),
        out_shape=jax.ShapeDtypeStruct((2, _R, _H), jnp.float32),
    )(x, W, b.reshape(1, _D))


def _stats_body(u_ref, i_ref, s1_ref, s2_ref):
    g = pl.program_id(0)

    @pl.when(g == 0)
    def _():
        s1_ref[...] = jnp.zeros_like(s1_ref)
        s2_ref[...] = jnp.zeros_like(s2_ref)

    xu = u_ref[...]
    xi = i_ref[...]
    s1_ref[...] += (jnp.sum(xu, axis=0) + jnp.sum(xi, axis=0)).reshape(1, _D)
    s2_ref[...] += (jnp.sum(xu * xu) + jnp.sum(xi * xi)).reshape(1, 1)


def _pairnorm_stats(u, i):
    BM = 5000
    return pl.pallas_call(
        _stats_body,
        grid=(_N // BM,),
        in_specs=[
            pl.BlockSpec((BM, _D), lambda g: (g, 0)),
            pl.BlockSpec((BM, _D), lambda g: (g, 0)),
        ],
        out_specs=[
            pl.BlockSpec((1, _D), lambda g: (0, 0)),
            pl.BlockSpec((1, 1), lambda g: (0, 0)),
        ],
        out_shape=[
            jax.ShapeDtypeStruct((1, _D), jnp.float32),
            jax.ShapeDtypeStruct((1, 1), jnp.float32),
        ],
    )(u, i)


def _pn_apply_body(u_ref, i_ref, s1_ref, s2_ref, uo_ref, io_ref):
    n = jnp.float32(2 * _N)
    mu = s1_ref[...] / n
    var = s2_ref[0, 0] / n - jnp.sum(mu * mu)
    inv = 1.0 / jnp.sqrt(var + 1e-6)
    ru = (u_ref[...] - mu) * inv
    ri = (i_ref[...] - mu) * inv
    uo_ref[0] = ru[:, :_H]
    uo_ref[1] = ru[:, _H:]
    io_ref[0] = ri[:, :_H]
    io_ref[1] = ri[:, _H:]


def _pairnorm_apply(u, i, s1, s2):
    BM = 1000
    return pl.pallas_call(
        _pn_apply_body,
        grid=(_N // BM,),
        in_specs=[
            pl.BlockSpec((BM, _D), lambda g: (g, 0)),
            pl.BlockSpec((BM, _D), lambda g: (g, 0)),
            pl.BlockSpec((1, _D), lambda g: (0, 0)),
            pl.BlockSpec((1, 1), lambda g: (0, 0)),
        ],
        out_specs=[
            pl.BlockSpec((2, BM, _H), lambda g: (0, g, 0)),
            pl.BlockSpec((2, BM, _H), lambda g: (0, g, 0)),
        ],
        out_shape=[
            jax.ShapeDtypeStruct((2, _R, _H), jnp.float32),
            jax.ShapeDtypeStruct((2, _R, _H), jnp.float32),
        ],
    )(u, i, s1, s2)


def _combine_body(e0_ref, e1_ref, e2_ref, f1_ref, f2_ref, o_ref):
    def nrm(x0, x1):
        n = jnp.sqrt(jnp.sum(x0 * x0 + x1 * x1, axis=1, keepdims=True))
        n = jnp.maximum(n, 1e-12)
        return x0 / n, x1 / n

    f1n0, f1n1 = nrm(f1_ref[0], f1_ref[1])
    f2n0, f2n1 = nrm(f2_ref[0], f2_ref[1])
    third = jnp.float32(1.0 / 3.0)
    o_ref[0] = ((e0_ref[0] + e1_ref[0] + e2_ref[0]) * third
                + 0.55 * f1n0 + 0.55 * f2n0)
    o_ref[1] = ((e0_ref[1] + e1_ref[1] + e2_ref[1]) * third
                + 0.55 * f1n1 + 0.55 * f2n1)


def _combine(e0, e1, e2, f1, f2):
    BM = 3136
    spec = pl.BlockSpec((2, BM, _H), lambda g: (0, g, 0))
    return pl.pallas_call(
        _combine_body,
        grid=(_R // BM,),
        in_specs=[spec] * 5,
        out_specs=spec,
        out_shape=jax.ShapeDtypeStruct((2, _R, _H), jnp.float32),
    )(e0, e1, e2, f1, f2)


_MESH = plsc.VectorSubcoreMesh(core_axis_name="c", subcore_axis_name="s")
_CPARAMS = pltpu.CompilerParams(use_tc_tiling_on_sc=False)


def _sc_spmm(dst, src, vals, xs, zeros):
    E = dst.shape[0]
    ept = E // 16
    nsup = ept // _SUP
    nx = len(xs)

    @functools.partial(
        pl.kernel,
        out_type=[jax.ShapeDtypeStruct((2, _R, _H), jnp.float32)] * nx,
        mesh=_MESH,
        compiler_params=_CPARAMS,
        scratch_types=[
            pltpu.VMEM((_SUP,), jnp.int32),
            pltpu.VMEM((_SUP,), jnp.int32),
            pltpu.VMEM((_SUP,), jnp.float32),
            pltpu.VMEM((_K,), jnp.int32),
            pltpu.VMEM((_K, _H), jnp.float32),
            pltpu.VMEM((_K, _H), jnp.float32),
            pltpu.VMEM_SHARED((_R, _H), jnp.float32),
            pltpu.SemaphoreType.DMA,
            pltpu.SemaphoreType.DMA,
        ],
    )
    def k(dst_hbm, src_hbm, val_hbm, *rest):
        x_hbms = rest[:nx]
        z_hbm = rest[nx]
        o_hbms = rest[nx + 1:2 * nx + 1]
        (dst_s, src_s, val_s, idx_w, rows0, rows1, acc,
         sem0, sem1) = rest[2 * nx + 1:]
        c = lax.axis_index("c")
        s = lax.axis_index("s")
        xoff = c * _R

        def sweep(x_hbm, o_hbm):
            pltpu.sync_copy(z_hbm, acc.at[pl.ds(s * _RPT, _RPT)])
            plsc.subcore_barrier()

            def issue(cb, buf, sem):
                pltpu.async_copy(x_hbm.at[src_s.at[pl.ds(cb, _K)]], buf, sem)

            def drain(buf, sem):
                pltpu.make_async_copy(x_hbm.at[pl.ds(0, _K)], buf, sem).wait()

            def process(cb, buf):
                for t in range(_K // 16):
                    idx_w[pl.ds(t * 16, 16)] = dst_s[pl.ds(cb + t * 16, 16)]

                def scale_blk(blk, cc):
                    v16 = val_s[pl.ds(cb + blk * 16, 16)]
                    row = blk * 16
                    for t in range(16):
                        sv = jnp.broadcast_to(v16[t], (16,))
                        for q in range(_H // 16):
                            buf[row + t, pl.ds(q * 16, 16)] = (
                                buf[row + t, pl.ds(q * 16, 16)] * sv)
                    return cc

                lax.fori_loop(0, _K // 16, scale_blk, 0)
                pltpu.sync_copy(buf, acc.at[idx_w], add=True)

            def super_body(sup, carry):
                ebase = s * ept + sup * _SUP
                pltpu.sync_copy(dst_hbm.at[pl.ds(ebase, _SUP)], dst_s)
                pltpu.sync_copy(src_hbm.at[pl.ds(ebase, _SUP)], src_s)
                pltpu.sync_copy(val_hbm.at[pl.ds(ebase, _SUP)], val_s)

                def loc(r, cc):
                    sl = pl.ds(r * 16, 16)
                    src_s[sl] = src_s[sl] + xoff
                    return cc

                lax.fori_loop(0, _SUP // 16, loc, 0)

                issue(0, rows0, sem0)

                def pair(p, cc):
                    c0 = 2 * p * _K
                    c1 = c0 + _K
                    i1 = 2 * p + 1

                    @pl.when(i1 < _NCH)
                    def _():
                        issue(c1, rows1, sem1)

                    drain(rows0, sem0)
                    process(c0, rows0)

                    @pl.when(i1 + 1 < _NCH)
                    def _():
                        issue(c1 + _K, rows0, sem0)

                    @pl.when(i1 < _NCH)
                    def _():
                        drain(rows1, sem1)
                        process(c1, rows1)

                    return cc

                lax.fori_loop(0, (_NCH + 1) // 2, pair, 0)
                return carry

            lax.fori_loop(0, nsup, super_body, 0)
            plsc.subcore_barrier()
            pltpu.sync_copy(acc.at[pl.ds(s * _RPT, _RPT)],
                            o_hbm.at[c, pl.ds(s * _RPT, _RPT)])
            plsc.subcore_barrier()

        for x_hbm, o_hbm in zip(x_hbms, o_hbms):
            sweep(x_hbm, o_hbm)

    outs = k(dst, src, vals, *xs, zeros)
    return list(outs) if isinstance(outs, (list, tuple)) else [outs]


def _flat(p):
    return jnp.reshape(p, (2 * _R, _H))


def kernel(ui_indices, ui_vals, iu_indices, iu_vals, user_id_emb, item_id_emb,
           image_feats, text_feats, W_img, b_img, W_txt, b_txt):
    img_f = _project(image_feats, W_img, b_img)
    txt_f = _project(text_feats, W_txt, b_txt)

    s1, s2 = _pairnorm_stats(user_id_emb, item_id_emb)
    u0, i0 = _pairnorm_apply(user_id_emb, item_id_emb, s1, s2)

    ui_dst, ui_src = ui_indices[0], ui_indices[1]
    iu_dst, iu_src = iu_indices[0], iu_indices[1]
    zeros = jnp.zeros((_RPT, _H), jnp.float32)

    img_u, txt_u, u1 = _sc_spmm(
        ui_dst, ui_src, ui_vals,
        [_flat(img_f), _flat(txt_f), _flat(i0)], zeros)
    img_i, txt_i, i1 = _sc_spmm(
        iu_dst, iu_src, iu_vals,
        [_flat(img_u), _flat(txt_u), _flat(u1)], zeros)
    (u2,) = _sc_spmm(ui_dst, ui_src, ui_vals, [_flat(i1)], zeros)
    (i2,) = _sc_spmm(iu_dst, iu_src, iu_vals, [_flat(u2)], zeros)

    u_fin = _combine(u0, u1, u2, img_u, txt_u)
    i_fin = _combine(i0, i1, i2, img_i, txt_i)
    u_final = jnp.concatenate([u_fin[0, :_N], u_fin[1, :_N]], axis=1)
    i_final = jnp.concatenate([i_fin[0, :_N], i_fin[1, :_N]], axis=1)
    return (u_final, i_final)


# speedup vs baseline: 6.2308x; 1.0901x over previous
import functools

import jax
import jax.numpy as jnp
from jax import lax
from jax.experimental import pallas as pl
from jax.experimental.pallas import tpu as pltpu
from jax.experimental.pallas import tpu_sc as plsc

_N = 50000
_D = 64
_H = 32
_R = 50176
_RPT = _R // 16
_K = 80
_SUP = 2000
_NCH = _SUP // _K


def _proj_body(x_ref, w_ref, b_ref, o_ref):
    res = (
        jnp.dot(x_ref[...], w_ref[...], preferred_element_type=jnp.float32)
        + b_ref[...]
    )
    o_ref[0] = res[:, :_H]
    o_ref[1] = res[:, _H:]


def _project(x, W, b):
    BM = 1000
    return pl.pallas_call(
        _proj_body,
        grid=(_N // BM,),
        in_specs=[
            pl.BlockSpec((BM, 128), lambda g: (g, 0)),
            pl.BlockSpec((128, _D), lambda g: (0, 0)),
            pl.BlockSpec((1, _D), lambda g: (0, 0)),
        ],
        out_specs=pl.BlockSpec((2, BM, _H), lambda g: (0, g, 0)),
        out_shape=jax.ShapeDtypeStruct((2, _R, _H), jnp.float32),
    )(x, W, b.reshape(1, _D))


def _stats_body(u_ref, i_ref, s1_ref, s2_ref):
    g = pl.program_id(0)

    @pl.when(g == 0)
    def _():
        s1_ref[...] = jnp.zeros_like(s1_ref)
        s2_ref[...] = jnp.zeros_like(s2_ref)

    xu = u_ref[...]
    xi = i_ref[...]
    s1_ref[...] += (jnp.sum(xu, axis=0) + jnp.sum(xi, axis=0)).reshape(1, _D)
    s2_ref[...] += (jnp.sum(xu * xu) + jnp.sum(xi * xi)).reshape(1, 1)


def _pairnorm_stats(u, i):
    BM = 5000
    return pl.pallas_call(
        _stats_body,
        grid=(_N // BM,),
        in_specs=[
            pl.BlockSpec((BM, _D), lambda g: (g, 0)),
            pl.BlockSpec((BM, _D), lambda g: (g, 0)),
        ],
        out_specs=[
            pl.BlockSpec((1, _D), lambda g: (0, 0)),
            pl.BlockSpec((1, 1), lambda g: (0, 0)),
        ],
        out_shape=[
            jax.ShapeDtypeStruct((1, _D), jnp.float32),
            jax.ShapeDtypeStruct((1, 1), jnp.float32),
        ],
    )(u, i)


def _pn_apply_body(u_ref, i_ref, s1_ref, s2_ref, uo_ref, io_ref):
    n = jnp.float32(2 * _N)
    mu = s1_ref[...] / n
    var = s2_ref[0, 0] / n - jnp.sum(mu * mu)
    inv = 1.0 / jnp.sqrt(var + 1e-6)
    ru = (u_ref[...] - mu) * inv
    ri = (i_ref[...] - mu) * inv
    uo_ref[0] = ru[:, :_H]
    uo_ref[1] = ru[:, _H:]
    io_ref[0] = ri[:, :_H]
    io_ref[1] = ri[:, _H:]


def _pairnorm_apply(u, i, s1, s2):
    BM = 1000
    return pl.pallas_call(
        _pn_apply_body,
        grid=(_N // BM,),
        in_specs=[
            pl.BlockSpec((BM, _D), lambda g: (g, 0)),
            pl.BlockSpec((BM, _D), lambda g: (g, 0)),
            pl.BlockSpec((1, _D), lambda g: (0, 0)),
            pl.BlockSpec((1, 1), lambda g: (0, 0)),
        ],
        out_specs=[
            pl.BlockSpec((2, BM, _H), lambda g: (0, g, 0)),
            pl.BlockSpec((2, BM, _H), lambda g: (0, g, 0)),
        ],
        out_shape=[
            jax.ShapeDtypeStruct((2, _R, _H), jnp.float32),
            jax.ShapeDtypeStruct((2, _R, _H), jnp.float32),
        ],
    )(u, i, s1, s2)


def _combine_body(e0_ref, e1_ref, e2_ref, f1_ref, f2_ref, o_ref):
    def nrm(x0, x1):
        n = jnp.sqrt(jnp.sum(x0 * x0 + x1 * x1, axis=1, keepdims=True))
        n = jnp.maximum(n, 1e-12)
        return x0 / n, x1 / n

    f1n0, f1n1 = nrm(f1_ref[0], f1_ref[1])
    f2n0, f2n1 = nrm(f2_ref[0], f2_ref[1])
    third = jnp.float32(1.0 / 3.0)
    o_ref[0] = ((e0_ref[0] + e1_ref[0] + e2_ref[0]) * third
                + 0.55 * f1n0 + 0.55 * f2n0)
    o_ref[1] = ((e0_ref[1] + e1_ref[1] + e2_ref[1]) * third
                + 0.55 * f1n1 + 0.55 * f2n1)


def _combine(e0, e1, e2, f1, f2):
    BM = 3136
    spec = pl.BlockSpec((2, BM, _H), lambda g: (0, g, 0))
    return pl.pallas_call(
        _combine_body,
        grid=(_R // BM,),
        in_specs=[spec] * 5,
        out_specs=spec,
        out_shape=jax.ShapeDtypeStruct((2, _R, _H), jnp.float32),
    )(e0, e1, e2, f1, f2)


_MESH = plsc.VectorSubcoreMesh(core_axis_name="c", subcore_axis_name="s")
_CPARAMS = pltpu.CompilerParams(use_tc_tiling_on_sc=False)


def _sc_spmm(dst, src, vals, xs, zeros):
    E = dst.shape[0]
    ept = E // 16
    nsup = ept // _SUP
    nx = len(xs)

    @functools.partial(
        pl.kernel,
        out_type=[jax.ShapeDtypeStruct((2, _R, _H), jnp.float32)] * nx,
        mesh=_MESH,
        compiler_params=_CPARAMS,
        scratch_types=[
            pltpu.VMEM((_SUP,), jnp.int32),
            pltpu.VMEM((_SUP,), jnp.int32),
            pltpu.VMEM((_SUP,), jnp.float32),
            pltpu.VMEM((_K,), jnp.int32),
            pltpu.VMEM((_K,), jnp.int32),
            pltpu.VMEM((_K,), jnp.int32),
            pltpu.VMEM((_K, _H), jnp.float32),
            pltpu.VMEM((_K, _H), jnp.float32),
            pltpu.VMEM((_K, _H), jnp.float32),
            pltpu.VMEM_SHARED((_R, _H), jnp.float32),
            pltpu.SemaphoreType.DMA,
            pltpu.SemaphoreType.DMA,
            pltpu.SemaphoreType.DMA,
            pltpu.SemaphoreType.DMA,
            pltpu.SemaphoreType.DMA,
            pltpu.SemaphoreType.DMA,
        ],
    )
    def k(dst_hbm, src_hbm, val_hbm, *rest):
        x_hbms = rest[:nx]
        z_hbm = rest[nx]
        o_hbms = rest[nx + 1:2 * nx + 1]
        (dst_s, src_s, val_s, idx0, idx1, idx2, rows0, rows1, rows2, acc,
         g0, g1, g2, t0, t1, t2) = rest[2 * nx + 1:]
        c = lax.axis_index("c")
        s = lax.axis_index("s")
        xoff = c * _R

        def sweep(x_hbm, o_hbm):
            pltpu.sync_copy(z_hbm, acc.at[pl.ds(s * _RPT, _RPT)])
            plsc.subcore_barrier()

            def issue(cb, buf, sem):
                pltpu.async_copy(x_hbm.at[src_s.at[pl.ds(cb, _K)]], buf, sem)

            def wdma(buf, sem):
                pltpu.make_async_copy(x_hbm.at[pl.ds(0, _K)], buf, sem).wait()

            def process(cb, buf, idx, ssem):
                for t in range(_K // 16):
                    idx[pl.ds(t * 16, 16)] = dst_s[pl.ds(cb + t * 16, 16)]

                def scale_blk(blk, cc):
                    v16 = val_s[pl.ds(cb + blk * 16, 16)]
                    row = blk * 16
                    for t in range(16):
                        sv = jnp.broadcast_to(v16[t], (16,))
                        for q in range(_H // 16):
                            buf[row + t, pl.ds(q * 16, 16)] = (
                                buf[row + t, pl.ds(q * 16, 16)] * sv)
                    return cc

                lax.fori_loop(0, _K // 16, scale_blk, 0)
                pltpu.async_copy(buf, acc.at[idx], ssem, add=True)

            def super_body(sup, carry):
                ebase = s * ept + sup * _SUP
                pltpu.sync_copy(dst_hbm.at[pl.ds(ebase, _SUP)], dst_s)
                pltpu.sync_copy(src_hbm.at[pl.ds(ebase, _SUP)], src_s)
                pltpu.sync_copy(val_hbm.at[pl.ds(ebase, _SUP)], val_s)

                def loc(r, cc):
                    sl = pl.ds(r * 16, 16)
                    src_s[sl] = src_s[sl] + xoff
                    return cc

                lax.fori_loop(0, _SUP // 16, loc, 0)

                issue(0, rows0, g0)

                def triple(t, cc):
                    i0 = 3 * t
                    i1, i2, i3 = i0 + 1, i0 + 2, i0 + 3

                    @pl.when((i1 < _NCH) & (i1 > 3))
                    def _():
                        wdma(rows1, t1)

                    @pl.when(i1 < _NCH)
                    def _():
                        issue(i1 * _K, rows1, g1)

                    wdma(rows0, g0)
                    process(i0 * _K, rows0, idx0, t0)

                    @pl.when((i2 < _NCH) & (i2 > 3))
                    def _():
                        wdma(rows2, t2)

                    @pl.when(i2 < _NCH)
                    def _():
                        issue(i2 * _K, rows2, g2)

                    @pl.when(i1 < _NCH)
                    def _():
                        wdma(rows1, g1)
                        process(i1 * _K, rows1, idx1, t1)

                    @pl.when(i3 < _NCH)
                    def _():
                        wdma(rows0, t0)
                        issue(i3 * _K, rows0, g0)

                    @pl.when(i2 < _NCH)
                    def _():
                        wdma(rows2, g2)
                        process(i2 * _K, rows2, idx2, t2)

                    return cc

                lax.fori_loop(0, (_NCH + 2) // 3, triple, 0)
                wdma(rows1, t1)
                wdma(rows2, t2)
                wdma(rows0, t0)
                return carry

            lax.fori_loop(0, nsup, super_body, 0)
            plsc.subcore_barrier()
            pltpu.sync_copy(acc.at[pl.ds(s * _RPT, _RPT)],
                            o_hbm.at[c, pl.ds(s * _RPT, _RPT)])
            plsc.subcore_barrier()

        for x_hbm, o_hbm in zip(x_hbms, o_hbms):
            sweep(x_hbm, o_hbm)

    outs = k(dst, src, vals, *xs, zeros)
    return list(outs) if isinstance(outs, (list, tuple)) else [outs]


def _flat(p):
    return jnp.reshape(p, (2 * _R, _H))


def kernel(ui_indices, ui_vals, iu_indices, iu_vals, user_id_emb, item_id_emb,
           image_feats, text_feats, W_img, b_img, W_txt, b_txt):
    img_f = _project(image_feats, W_img, b_img)
    txt_f = _project(text_feats, W_txt, b_txt)

    s1, s2 = _pairnorm_stats(user_id_emb, item_id_emb)
    u0, i0 = _pairnorm_apply(user_id_emb, item_id_emb, s1, s2)

    ui_dst, ui_src = ui_indices[0], ui_indices[1]
    iu_dst, iu_src = iu_indices[0], iu_indices[1]
    zeros = jnp.zeros((_RPT, _H), jnp.float32)

    img_u, txt_u, u1 = _sc_spmm(
        ui_dst, ui_src, ui_vals,
        [_flat(img_f), _flat(txt_f), _flat(i0)], zeros)
    img_i, txt_i, i1 = _sc_spmm(
        iu_dst, iu_src, iu_vals,
        [_flat(img_u), _flat(txt_u), _flat(u1)], zeros)
    (u2,) = _sc_spmm(ui_dst, ui_src, ui_vals, [_flat(i1)], zeros)
    (i2,) = _sc_spmm(iu_dst, iu_src, iu_vals, [_flat(u2)], zeros)

    u_fin = _combine(u0, u1, u2, img_u, txt_u)
    i_fin = _combine(i0, i1, i2, img_i, txt_i)
    u_final = jnp.concatenate([u_fin[0, :_N], u_fin[1, :_N]], axis=1)
    i_final = jnp.concatenate([i_fin[0, :_N], i_fin[1, :_N]], axis=1)
    return (u_final, i_final)


# speedup vs baseline: 7.2275x; 1.1600x over previous
import functools

import jax
import jax.numpy as jnp
from jax import lax
from jax.experimental import pallas as pl
from jax.experimental.pallas import tpu as pltpu
from jax.experimental.pallas import tpu_sc as plsc

_N = 50000
_D = 64
_H = 32
_R = 50176
_RPT = _R // 16
_K = 128
_SUP = 2048
_NCH = _SUP // _K


def _proj_body(x_ref, w_ref, b_ref, o_ref):
    res = (
        jnp.dot(x_ref[...], w_ref[...], preferred_element_type=jnp.float32)
        + b_ref[...]
    )
    o_ref[0] = res[:, :_H]
    o_ref[1] = res[:, _H:]


def _project(x, W, b):
    BM = 1000
    return pl.pallas_call(
        _proj_body,
        grid=(_N // BM,),
        in_specs=[
            pl.BlockSpec((BM, 128), lambda g: (g, 0)),
            pl.BlockSpec((128, _D), lambda g: (0, 0)),
            pl.BlockSpec((1, _D), lambda g: (0, 0)),
        ],
        out_specs=pl.BlockSpec((2, BM, _H), lambda g: (0, g, 0)),
        out_shape=jax.ShapeDtypeStruct((2, _R, _H), jnp.float32),
    )(x, W, b.reshape(1, _D))


def _stats_body(u_ref, i_ref, s1_ref, s2_ref):
    g = pl.program_id(0)

    @pl.when(g == 0)
    def _():
        s1_ref[...] = jnp.zeros_like(s1_ref)
        s2_ref[...] = jnp.zeros_like(s2_ref)

    xu = u_ref[...]
    xi = i_ref[...]
    s1_ref[...] += (jnp.sum(xu, axis=0) + jnp.sum(xi, axis=0)).reshape(1, _D)
    s2_ref[...] += (jnp.sum(xu * xu) + jnp.sum(xi * xi)).reshape(1, 1)


def _pairnorm_stats(u, i):
    BM = 5000
    return pl.pallas_call(
        _stats_body,
        grid=(_N // BM,),
        in_specs=[
            pl.BlockSpec((BM, _D), lambda g: (g, 0)),
            pl.BlockSpec((BM, _D), lambda g: (g, 0)),
        ],
        out_specs=[
            pl.BlockSpec((1, _D), lambda g: (0, 0)),
            pl.BlockSpec((1, 1), lambda g: (0, 0)),
        ],
        out_shape=[
            jax.ShapeDtypeStruct((1, _D), jnp.float32),
            jax.ShapeDtypeStruct((1, 1), jnp.float32),
        ],
    )(u, i)


def _pn_apply_body(u_ref, i_ref, s1_ref, s2_ref, uo_ref, io_ref):
    n = jnp.float32(2 * _N)
    mu = s1_ref[...] / n
    var = s2_ref[0, 0] / n - jnp.sum(mu * mu)
    inv = 1.0 / jnp.sqrt(var + 1e-6)
    ru = (u_ref[...] - mu) * inv
    ri = (i_ref[...] - mu) * inv
    uo_ref[0] = ru[:, :_H]
    uo_ref[1] = ru[:, _H:]
    io_ref[0] = ri[:, :_H]
    io_ref[1] = ri[:, _H:]


def _pairnorm_apply(u, i, s1, s2):
    BM = 1000
    return pl.pallas_call(
        _pn_apply_body,
        grid=(_N // BM,),
        in_specs=[
            pl.BlockSpec((BM, _D), lambda g: (g, 0)),
            pl.BlockSpec((BM, _D), lambda g: (g, 0)),
            pl.BlockSpec((1, _D), lambda g: (0, 0)),
            pl.BlockSpec((1, 1), lambda g: (0, 0)),
        ],
        out_specs=[
            pl.BlockSpec((2, BM, _H), lambda g: (0, g, 0)),
            pl.BlockSpec((2, BM, _H), lambda g: (0, g, 0)),
        ],
        out_shape=[
            jax.ShapeDtypeStruct((2, _R, _H), jnp.float32),
            jax.ShapeDtypeStruct((2, _R, _H), jnp.float32),
        ],
    )(u, i, s1, s2)


def _combine_body(e0_ref, e1_ref, e2_ref, f1_ref, f2_ref, o_ref):
    def nrm(x0, x1):
        n = jnp.sqrt(jnp.sum(x0 * x0 + x1 * x1, axis=1, keepdims=True))
        n = jnp.maximum(n, 1e-12)
        return x0 / n, x1 / n

    f1n0, f1n1 = nrm(f1_ref[0], f1_ref[1])
    f2n0, f2n1 = nrm(f2_ref[0], f2_ref[1])
    third = jnp.float32(1.0 / 3.0)
    o_ref[0] = ((e0_ref[0] + e1_ref[0] + e2_ref[0]) * third
                + 0.55 * f1n0 + 0.55 * f2n0)
    o_ref[1] = ((e0_ref[1] + e1_ref[1] + e2_ref[1]) * third
                + 0.55 * f1n1 + 0.55 * f2n1)


def _combine(e0, e1, e2, f1, f2):
    BM = 3136
    spec = pl.BlockSpec((2, BM, _H), lambda g: (0, g, 0))
    return pl.pallas_call(
        _combine_body,
        grid=(_R // BM,),
        in_specs=[spec] * 5,
        out_specs=spec,
        out_shape=jax.ShapeDtypeStruct((2, _R, _H), jnp.float32),
    )(e0, e1, e2, f1, f2)


_MESH = plsc.VectorSubcoreMesh(core_axis_name="c", subcore_axis_name="s")
_CPARAMS = pltpu.CompilerParams(use_tc_tiling_on_sc=False)


def _sc_spmm(dst, src, vals, xs, zeros):
    E = dst.shape[0]
    nsupg = (E + _SUP - 1) // _SUP
    laste = E - (nsupg - 1) * _SUP
    lastc = laste // _K
    assert laste % _K == 0 and _NCH % 3 == 1 and lastc % 3 == 1
    jmax = (nsupg + 15) // 16
    nx = len(xs)

    @functools.partial(
        pl.kernel,
        out_type=[jax.ShapeDtypeStruct((2, _R, _H), jnp.float32)] * nx,
        mesh=_MESH,
        compiler_params=_CPARAMS,
        scratch_types=[
            pltpu.VMEM((_SUP,), jnp.int32),
            pltpu.VMEM((_SUP,), jnp.int32),
            pltpu.VMEM((_SUP,), jnp.float32),
            pltpu.VMEM((_K,), jnp.int32),
            pltpu.VMEM((_K,), jnp.int32),
            pltpu.VMEM((_K,), jnp.int32),
            pltpu.VMEM((_K, _H), jnp.float32),
            pltpu.VMEM((_K, _H), jnp.float32),
            pltpu.VMEM((_K, _H), jnp.float32),
            pltpu.VMEM_SHARED((_R, _H), jnp.float32),
            pltpu.SemaphoreType.DMA,
            pltpu.SemaphoreType.DMA,
            pltpu.SemaphoreType.DMA,
            pltpu.SemaphoreType.DMA,
            pltpu.SemaphoreType.DMA,
            pltpu.SemaphoreType.DMA,
        ],
    )
    def k(dst_hbm, src_hbm, val_hbm, *rest):
        x_hbms = rest[:nx]
        z_hbm = rest[nx]
        o_hbms = rest[nx + 1:2 * nx + 1]
        (dst_s, src_s, val_s, idx0, idx1, idx2, rows0, rows1, rows2, acc,
         g0, g1, g2, t0, t1, t2) = rest[2 * nx + 1:]
        c = lax.axis_index("c")
        s = lax.axis_index("s")
        xoff = c * _R

        def sweep(x_hbm, o_hbm):
            pltpu.sync_copy(z_hbm, acc.at[pl.ds(s * _RPT, _RPT)])
            plsc.subcore_barrier()

            def issue(cb, buf, sem):
                pltpu.async_copy(x_hbm.at[src_s.at[pl.ds(cb, _K)]], buf, sem)

            def wdma(buf, sem):
                pltpu.make_async_copy(x_hbm.at[pl.ds(0, _K)], buf, sem).wait()

            def process(cb, buf, idx, ssem):
                for t in range(_K // 16):
                    idx[pl.ds(t * 16, 16)] = dst_s[pl.ds(cb + t * 16, 16)]

                def scale_blk(blk, cc):
                    v16 = val_s[pl.ds(cb + blk * 16, 16)]
                    row = blk * 16
                    for t in range(16):
                        sv = jnp.broadcast_to(v16[t], (16,))
                        for q in range(_H // 16):
                            buf[row + t, pl.ds(q * 16, 16)] = (
                                buf[row + t, pl.ds(q * 16, 16)] * sv)
                    return cc

                lax.fori_loop(0, _K // 16, scale_blk, 0)
                pltpu.async_copy(buf, acc.at[idx], ssem, add=True)

            def super_body(j, carry):
                sup = s + 16 * j

                @pl.when(sup < nsupg)
                def _do_super():
                    last = sup == (nsupg - 1)
                    ebase = sup * _SUP

                    @pl.when(jnp.logical_not(last))
                    def _():
                        pltpu.sync_copy(dst_hbm.at[pl.ds(ebase, _SUP)], dst_s)
                        pltpu.sync_copy(src_hbm.at[pl.ds(ebase, _SUP)], src_s)
                        pltpu.sync_copy(val_hbm.at[pl.ds(ebase, _SUP)], val_s)

                    @pl.when(last)
                    def _():
                        pltpu.sync_copy(dst_hbm.at[pl.ds(ebase, laste)],
                                        dst_s.at[pl.ds(0, laste)])
                        pltpu.sync_copy(src_hbm.at[pl.ds(ebase, laste)],
                                        src_s.at[pl.ds(0, laste)])
                        pltpu.sync_copy(val_hbm.at[pl.ds(ebase, laste)],
                                        val_s.at[pl.ds(0, laste)])

                    nch = jnp.where(last, lastc, _NCH)

                    def loc(r, cc):
                        sl = pl.ds(r * 16, 16)
                        src_s[sl] = src_s[sl] + xoff
                        return cc

                    lax.fori_loop(0, _SUP // 16, loc, 0)

                    issue(0, rows0, g0)

                    def triple(t, cc):
                        i0 = 3 * t
                        i1, i2, i3 = i0 + 1, i0 + 2, i0 + 3

                        @pl.when((i1 < nch) & (i1 > 3))
                        def _():
                            wdma(rows1, t1)

                        @pl.when(i1 < nch)
                        def _():
                            issue(i1 * _K, rows1, g1)

                        wdma(rows0, g0)
                        process(i0 * _K, rows0, idx0, t0)

                        @pl.when((i2 < nch) & (i2 > 3))
                        def _():
                            wdma(rows2, t2)

                        @pl.when(i2 < nch)
                        def _():
                            issue(i2 * _K, rows2, g2)

                        @pl.when(i1 < nch)
                        def _():
                            wdma(rows1, g1)
                            process(i1 * _K, rows1, idx1, t1)

                        @pl.when(i3 < nch)
                        def _():
                            wdma(rows0, t0)
                            issue(i3 * _K, rows0, g0)

                        @pl.when(i2 < nch)
                        def _():
                            wdma(rows2, g2)
                            process(i2 * _K, rows2, idx2, t2)

                        return cc

                    lax.fori_loop(0, (nch + 2) // 3, triple, 0)
                    wdma(rows1, t1)
                    wdma(rows2, t2)
                    wdma(rows0, t0)

                return carry

            lax.fori_loop(0, jmax, super_body, 0)
            plsc.subcore_barrier()
            pltpu.sync_copy(acc.at[pl.ds(s * _RPT, _RPT)],
                            o_hbm.at[c, pl.ds(s * _RPT, _RPT)])
            plsc.subcore_barrier()

        for x_hbm, o_hbm in zip(x_hbms, o_hbms):
            sweep(x_hbm, o_hbm)

    outs = k(dst, src, vals, *xs, zeros)
    return list(outs) if isinstance(outs, (list, tuple)) else [outs]


def _flat(p):
    return jnp.reshape(p, (2 * _R, _H))


def kernel(ui_indices, ui_vals, iu_indices, iu_vals, user_id_emb, item_id_emb,
           image_feats, text_feats, W_img, b_img, W_txt, b_txt):
    img_f = _project(image_feats, W_img, b_img)
    txt_f = _project(text_feats, W_txt, b_txt)

    s1, s2 = _pairnorm_stats(user_id_emb, item_id_emb)
    u0, i0 = _pairnorm_apply(user_id_emb, item_id_emb, s1, s2)

    ui_dst, ui_src = ui_indices[0], ui_indices[1]
    iu_dst, iu_src = iu_indices[0], iu_indices[1]
    zeros = jnp.zeros((_RPT, _H), jnp.float32)

    img_u, txt_u, u1 = _sc_spmm(
        ui_dst, ui_src, ui_vals,
        [_flat(img_f), _flat(txt_f), _flat(i0)], zeros)
    img_i, txt_i, i1 = _sc_spmm(
        iu_dst, iu_src, iu_vals,
        [_flat(img_u), _flat(txt_u), _flat(u1)], zeros)
    (u2,) = _sc_spmm(ui_dst, ui_src, ui_vals, [_flat(i1)], zeros)
    (i2,) = _sc_spmm(iu_dst, iu_src, iu_vals, [_flat(u2)], zeros)

    u_fin = _combine(u0, u1, u2, img_u, txt_u)
    i_fin = _combine(i0, i1, i2, img_i, txt_i)
    u_final = jnp.concatenate([u_fin[0, :_N], u_fin[1, :_N]], axis=1)
    i_final = jnp.concatenate([i_fin[0, :_N], i_fin[1, :_N]], axis=1)
    return (u_final, i_final)


# speedup vs baseline: 7.7485x; 1.0721x over previous
import functools

import jax
import jax.numpy as jnp
from jax import lax
from jax.experimental import pallas as pl
from jax.experimental.pallas import tpu as pltpu
from jax.experimental.pallas import tpu_sc as plsc

_N = 50000
_D = 64
_H = 32
_R = 50176
_RPT = _R // 16
_K = 128
_SUP = 2048
_NCH = _SUP // _K


def _proj_body(x_ref, w_ref, b_ref, o_ref):
    res = (
        jnp.dot(x_ref[...], w_ref[...], preferred_element_type=jnp.float32)
        + b_ref[...]
    )
    o_ref[0] = res[:, :_H]
    o_ref[1] = res[:, _H:]


def _project(x, W, b):
    BM = 1000
    return pl.pallas_call(
        _proj_body,
        grid=(_N // BM,),
        in_specs=[
            pl.BlockSpec((BM, 128), lambda g: (g, 0)),
            pl.BlockSpec((128, _D), lambda g: (0, 0)),
            pl.BlockSpec((1, _D), lambda g: (0, 0)),
        ],
        out_specs=pl.BlockSpec((2, BM, _H), lambda g: (0, g, 0)),
        out_shape=jax.ShapeDtypeStruct((2, _R, _H), jnp.float32),
    )(x, W, b.reshape(1, _D))


def _stats_body(u_ref, i_ref, s1_ref, s2_ref):
    g = pl.program_id(0)

    @pl.when(g == 0)
    def _():
        s1_ref[...] = jnp.zeros_like(s1_ref)
        s2_ref[...] = jnp.zeros_like(s2_ref)

    xu = u_ref[...]
    xi = i_ref[...]
    s1_ref[...] += (jnp.sum(xu, axis=0) + jnp.sum(xi, axis=0)).reshape(1, _D)
    s2_ref[...] += (jnp.sum(xu * xu) + jnp.sum(xi * xi)).reshape(1, 1)


def _pairnorm_stats(u, i):
    BM = 5000
    return pl.pallas_call(
        _stats_body,
        grid=(_N // BM,),
        in_specs=[
            pl.BlockSpec((BM, _D), lambda g: (g, 0)),
            pl.BlockSpec((BM, _D), lambda g: (g, 0)),
        ],
        out_specs=[
            pl.BlockSpec((1, _D), lambda g: (0, 0)),
            pl.BlockSpec((1, 1), lambda g: (0, 0)),
        ],
        out_shape=[
            jax.ShapeDtypeStruct((1, _D), jnp.float32),
            jax.ShapeDtypeStruct((1, 1), jnp.float32),
        ],
    )(u, i)


def _pn_apply_body(u_ref, i_ref, s1_ref, s2_ref, uo_ref, io_ref):
    n = jnp.float32(2 * _N)
    mu = s1_ref[...] / n
    var = s2_ref[0, 0] / n - jnp.sum(mu * mu)
    inv = 1.0 / jnp.sqrt(var + 1e-6)
    ru = (u_ref[...] - mu) * inv
    ri = (i_ref[...] - mu) * inv
    uo_ref[0] = ru[:, :_H]
    uo_ref[1] = ru[:, _H:]
    io_ref[0] = ri[:, :_H]
    io_ref[1] = ri[:, _H:]


def _pairnorm_apply(u, i, s1, s2):
    BM = 1000
    return pl.pallas_call(
        _pn_apply_body,
        grid=(_N // BM,),
        in_specs=[
            pl.BlockSpec((BM, _D), lambda g: (g, 0)),
            pl.BlockSpec((BM, _D), lambda g: (g, 0)),
            pl.BlockSpec((1, _D), lambda g: (0, 0)),
            pl.BlockSpec((1, 1), lambda g: (0, 0)),
        ],
        out_specs=[
            pl.BlockSpec((2, BM, _H), lambda g: (0, g, 0)),
            pl.BlockSpec((2, BM, _H), lambda g: (0, g, 0)),
        ],
        out_shape=[
            jax.ShapeDtypeStruct((2, _R, _H), jnp.float32),
            jax.ShapeDtypeStruct((2, _R, _H), jnp.float32),
        ],
    )(u, i, s1, s2)


def _combine_body(e0_ref, e1_ref, e2_ref, f1_ref, f2_ref, o_ref):
    def nrm(x0, x1):
        n = jnp.sqrt(jnp.sum(x0 * x0 + x1 * x1, axis=1, keepdims=True))
        n = jnp.maximum(n, 1e-12)
        return x0 / n, x1 / n

    f1n0, f1n1 = nrm(f1_ref[0], f1_ref[1])
    f2n0, f2n1 = nrm(f2_ref[0], f2_ref[1])
    third = jnp.float32(1.0 / 3.0)
    o_ref[0] = ((e0_ref[0] + e1_ref[0] + e2_ref[0]) * third
                + 0.55 * f1n0 + 0.55 * f2n0)
    o_ref[1] = ((e0_ref[1] + e1_ref[1] + e2_ref[1]) * third
                + 0.55 * f1n1 + 0.55 * f2n1)


def _combine(e0, e1, e2, f1, f2):
    BM = 3136
    spec = pl.BlockSpec((2, BM, _H), lambda g: (0, g, 0))
    return pl.pallas_call(
        _combine_body,
        grid=(_R // BM,),
        in_specs=[spec] * 5,
        out_specs=spec,
        out_shape=jax.ShapeDtypeStruct((2, _R, _H), jnp.float32),
    )(e0, e1, e2, f1, f2)


_MESH = plsc.VectorSubcoreMesh(core_axis_name="c", subcore_axis_name="s")
_CPARAMS = pltpu.CompilerParams(use_tc_tiling_on_sc=False)


def _sc_spmm(dst, src, vals, xs, zeros):
    E = dst.shape[0]
    nsupg = (E + _SUP - 1) // _SUP
    laste = E - (nsupg - 1) * _SUP
    lastc = laste // _K
    assert laste % _K == 0 and _NCH % 3 == 1 and lastc % 3 == 1
    jmax = (nsupg + 15) // 16
    nx = len(xs)

    @functools.partial(
        pl.kernel,
        out_type=[jax.ShapeDtypeStruct((2, _R, _H), jnp.float32)] * nx,
        mesh=_MESH,
        compiler_params=_CPARAMS,
        scratch_types=[
            pltpu.VMEM((_SUP,), jnp.int32),
            pltpu.VMEM((_SUP,), jnp.int32),
            pltpu.VMEM((_SUP,), jnp.float32),
            pltpu.VMEM((_K,), jnp.int32),
            pltpu.VMEM((_K,), jnp.int32),
            pltpu.VMEM((_K,), jnp.int32),
            pltpu.VMEM((_K, _H), jnp.float32),
            pltpu.VMEM((_K, _H), jnp.float32),
            pltpu.VMEM((_K, _H), jnp.float32),
            pltpu.VMEM_SHARED((_R, _H), jnp.float32),
            pltpu.SemaphoreType.DMA,
            pltpu.SemaphoreType.DMA,
            pltpu.SemaphoreType.DMA,
            pltpu.SemaphoreType.DMA,
            pltpu.SemaphoreType.DMA,
            pltpu.SemaphoreType.DMA,
        ],
    )
    def k(dst_hbm, src_hbm, val_hbm, *rest):
        x_hbms = rest[:nx]
        z_hbm = rest[nx]
        o_hbms = rest[nx + 1:2 * nx + 1]
        (dst_s, src_s, val_s, idx0, idx1, idx2, rows0, rows1, rows2, acc,
         g0, g1, g2, t0, t1, t2) = rest[2 * nx + 1:]
        c = lax.axis_index("c")
        s = lax.axis_index("s")
        xoff = c * _R

        def sweep(x_hbm, o_hbm):
            pltpu.sync_copy(z_hbm, acc.at[pl.ds(s * _RPT, _RPT)])
            plsc.subcore_barrier()

            def issue(cb, buf, sem):
                pltpu.async_copy(x_hbm.at[src_s.at[pl.ds(cb, _K)]], buf, sem)

            def wdma(buf, sem):
                pltpu.make_async_copy(x_hbm.at[pl.ds(0, _K)], buf, sem).wait()

            def process(cb, buf, idx, ssem):
                for t in range(_K // 16):
                    idx[pl.ds(t * 16, 16)] = dst_s[pl.ds(cb + t * 16, 16)]

                def scale_blk(blk, cc):
                    v16 = val_s[pl.ds(cb + blk * 16, 16)]
                    row = blk * 16
                    for t in range(16):
                        sv = jnp.broadcast_to(v16[t], (16,))
                        for q in range(_H // 16):
                            buf[row + t, pl.ds(q * 16, 16)] = (
                                buf[row + t, pl.ds(q * 16, 16)] * sv)
                    return cc

                lax.fori_loop(0, _K // 16, scale_blk, 0)
                pltpu.async_copy(buf, acc.at[idx], ssem, add=True)

            def super_body(j, carry):
                sup = s + 16 * j

                @pl.when(sup < nsupg)
                def _do_super():
                    last = sup == (nsupg - 1)
                    ebase = sup * _SUP

                    @pl.when(jnp.logical_not(last))
                    def _():
                        pltpu.async_copy(dst_hbm.at[pl.ds(ebase, _SUP)],
                                         dst_s, g0)
                        pltpu.async_copy(src_hbm.at[pl.ds(ebase, _SUP)],
                                         src_s, g1)
                        pltpu.async_copy(val_hbm.at[pl.ds(ebase, _SUP)],
                                         val_s, g2)
                        pltpu.make_async_copy(
                            dst_hbm.at[pl.ds(0, _SUP)], dst_s, g0).wait()
                        pltpu.make_async_copy(
                            src_hbm.at[pl.ds(0, _SUP)], src_s, g1).wait()
                        pltpu.make_async_copy(
                            val_hbm.at[pl.ds(0, _SUP)], val_s, g2).wait()

                    @pl.when(last)
                    def _():
                        pltpu.async_copy(dst_hbm.at[pl.ds(ebase, laste)],
                                         dst_s.at[pl.ds(0, laste)], g0)
                        pltpu.async_copy(src_hbm.at[pl.ds(ebase, laste)],
                                         src_s.at[pl.ds(0, laste)], g1)
                        pltpu.async_copy(val_hbm.at[pl.ds(ebase, laste)],
                                         val_s.at[pl.ds(0, laste)], g2)
                        pltpu.make_async_copy(
                            dst_hbm.at[pl.ds(0, laste)],
                            dst_s.at[pl.ds(0, laste)], g0).wait()
                        pltpu.make_async_copy(
                            src_hbm.at[pl.ds(0, laste)],
                            src_s.at[pl.ds(0, laste)], g1).wait()
                        pltpu.make_async_copy(
                            val_hbm.at[pl.ds(0, laste)],
                            val_s.at[pl.ds(0, laste)], g2).wait()

                    nch = jnp.where(last, lastc, _NCH)

                    def loc(r, cc):
                        sl = pl.ds(r * 16, 16)
                        src_s[sl] = src_s[sl] + xoff
                        return cc

                    lax.fori_loop(0, _SUP // 16, loc, 0)

                    issue(0, rows0, g0)

                    def triple(t, cc):
                        i0 = 3 * t
                        i1, i2, i3 = i0 + 1, i0 + 2, i0 + 3

                        @pl.when((i1 < nch) & (i1 > 3))
                        def _():
                            wdma(rows1, t1)

                        @pl.when(i1 < nch)
                        def _():
                            issue(i1 * _K, rows1, g1)

                        wdma(rows0, g0)
                        process(i0 * _K, rows0, idx0, t0)

                        @pl.when((i2 < nch) & (i2 > 3))
                        def _():
                            wdma(rows2, t2)

                        @pl.when(i2 < nch)
                        def _():
                            issue(i2 * _K, rows2, g2)

                        @pl.when(i1 < nch)
                        def _():
                            wdma(rows1, g1)
                            process(i1 * _K, rows1, idx1, t1)

                        @pl.when(i3 < nch)
                        def _():
                            wdma(rows0, t0)
                            issue(i3 * _K, rows0, g0)

                        @pl.when(i2 < nch)
                        def _():
                            wdma(rows2, g2)
                            process(i2 * _K, rows2, idx2, t2)

                        return cc

                    lax.fori_loop(0, (nch + 2) // 3, triple, 0)
                    wdma(rows1, t1)
                    wdma(rows2, t2)
                    wdma(rows0, t0)

                return carry

            lax.fori_loop(0, jmax, super_body, 0)
            plsc.subcore_barrier()
            pltpu.sync_copy(acc.at[pl.ds(s * _RPT, _RPT)],
                            o_hbm.at[c, pl.ds(s * _RPT, _RPT)])
            plsc.subcore_barrier()

        for x_hbm, o_hbm in zip(x_hbms, o_hbms):
            sweep(x_hbm, o_hbm)

    outs = k(dst, src, vals, *xs, zeros)
    return list(outs) if isinstance(outs, (list, tuple)) else [outs]


def _flat(p):
    return jnp.reshape(p, (2 * _R, _H))


def kernel(ui_indices, ui_vals, iu_indices, iu_vals, user_id_emb, item_id_emb,
           image_feats, text_feats, W_img, b_img, W_txt, b_txt):
    img_f = _project(image_feats, W_img, b_img)
    txt_f = _project(text_feats, W_txt, b_txt)

    s1, s2 = _pairnorm_stats(user_id_emb, item_id_emb)
    u0, i0 = _pairnorm_apply(user_id_emb, item_id_emb, s1, s2)

    ui_dst, ui_src = ui_indices[0], ui_indices[1]
    iu_dst, iu_src = iu_indices[0], iu_indices[1]
    zeros = jnp.zeros((_RPT, _H), jnp.float32)

    img_u, txt_u, u1 = _sc_spmm(
        ui_dst, ui_src, ui_vals,
        [_flat(img_f), _flat(txt_f), _flat(i0)], zeros)
    img_i, txt_i, i1 = _sc_spmm(
        iu_dst, iu_src, iu_vals,
        [_flat(img_u), _flat(txt_u), _flat(u1)], zeros)
    (u2,) = _sc_spmm(ui_dst, ui_src, ui_vals, [_flat(i1)], zeros)
    (i2,) = _sc_spmm(iu_dst, iu_src, iu_vals, [_flat(u2)], zeros)

    u_fin = _combine(u0, u1, u2, img_u, txt_u)
    i_fin = _combine(i0, i1, i2, img_i, txt_i)
    u_final = jnp.concatenate([u_fin[0, :_N], u_fin[1, :_N]], axis=1)
    i_final = jnp.concatenate([i_fin[0, :_N], i_fin[1, :_N]], axis=1)
    return (u_final, i_final)
